# baseline (speedup 1.0000x reference)
import functools

import jax
import jax.numpy as jnp
from jax import lax
from jax.experimental import pallas as pl
from jax.experimental.pallas import tpu as pltpu
from jax.experimental.pallas import tpu_sc as plsc

N_ATOMS = 100000
EMB = 128
VOCAB = 100
CHUNK = 128
NC = 2
NS = 16
NW = NC * NS
L = 16

_N_CHUNKS = -(-N_ATOMS // CHUNK)
_LAST_BASE = N_ATOMS - CHUNK
_SLOTS = -(-_N_CHUNKS // NW)
_FULL_W = _N_CHUNKS - (_SLOTS - 1) * NW
_DEPTH = 6


@functools.partial(
    pl.kernel,
    mesh=plsc.VectorSubcoreMesh(core_axis_name="c", subcore_axis_name="s"),
    out_type=jax.ShapeDtypeStruct((N_ATOMS, EMB), jnp.float32),
    scratch_types=[
        pltpu.VMEM((_SLOTS, CHUNK), jnp.int32),
        [pltpu.VMEM((CHUNK, EMB), jnp.float32)] * _DEPTH,
        pltpu.VMEM_SHARED((VOCAB, EMB), jnp.float32),
        pltpu.SemaphoreType.DMA,
        [pltpu.SemaphoreType.DMA] * _DEPTH,
        [pltpu.SemaphoreType.DMA] * _DEPTH,
    ],
)
def _emb_kernel(z_hbm, w_hbm, out_hbm, idx_v, rows, w_sh, isem, gsem, ssem):
    wid = lax.axis_index("s") * NC + lax.axis_index("c")

    @pl.when(lax.axis_index("s") == 0)
    def _():
        pltpu.sync_copy(w_hbm, w_sh)

    plsc.subcore_barrier()

    def base_of(k):
        c = jnp.minimum(wid + k * NW, _N_CHUNKS - 1)
        return pl.multiple_of(jnp.minimum(c * CHUNK, _LAST_BASE), 8)

    def when_owned(k, fn):
        if k == _SLOTS - 1:
            pl.when(wid < _FULL_W)(fn)
        else:
            fn()

    def prefetch(k):
        pltpu.async_copy(z_hbm.at[pl.ds(base_of(k), CHUNK)], idx_v.at[k],
                         isem)

    for k in range(_SLOTS):
        when_owned(k, functools.partial(prefetch, k))

    def shift(k):
        pltpu.make_async_copy(z_hbm.at[pl.ds(base_of(k), CHUNK)],
                              idx_v.at[k], isem).wait()
        for j in range(CHUNK // L):
            sl = pl.ds(j * L, L)
            idx_v[k, sl] = idx_v[k, sl] - 1

    for k in range(_SLOTS):
        when_owned(k, functools.partial(shift, k))

    def gather(k):
        b = k % _DEPTH
        pltpu.async_copy(w_sh.at[idx_v.at[k]], rows[b], gsem[b])

    def scatter(k):
        b = k % _DEPTH
        pltpu.make_async_copy(w_sh.at[idx_v.at[k]], rows[b],
                              gsem[b]).wait()
        pltpu.async_copy(rows[b], out_hbm.at[pl.ds(base_of(k), CHUNK)],
                         ssem[b])

    def wait_scatter(k):
        b = k % _DEPTH
        pltpu.make_async_copy(rows[b], out_hbm.at[pl.ds(base_of(k), CHUNK)],
                              ssem[b]).wait()

    for k in range(_SLOTS):
        if k >= _DEPTH:
            wait_scatter(k - _DEPTH)
        when_owned(k, functools.partial(gather, k))
        if k >= 1:
            when_owned(k - 1, functools.partial(scatter, k - 1))
    when_owned(_SLOTS - 1, functools.partial(scatter, _SLOTS - 1))
    for k in range(_SLOTS - _DEPTH, _SLOTS):
        when_owned(k, functools.partial(wait_scatter, k))


def kernel(Z, W):
    return _emb_kernel(Z, W)


# speedup vs baseline: 1.0071x; 1.0071x over previous
import functools

import jax
import jax.numpy as jnp
from jax import lax
from jax.experimental import pallas as pl
from jax.experimental.pallas import tpu as pltpu
from jax.experimental.pallas import tpu_sc as plsc

N_ATOMS = 100000
EMB = 128
VOCAB = 100
CHUNK = 128
NC = 2
NS = 16
NW = NC * NS
L = 16

_N_CHUNKS = -(-N_ATOMS // CHUNK)
_LAST_BASE = N_ATOMS - CHUNK
_SLOTS = -(-_N_CHUNKS // NW)
_FULL_W = _N_CHUNKS - (_SLOTS - 1) * NW
_DEPTH = 5


@functools.partial(
    pl.kernel,
    mesh=plsc.VectorSubcoreMesh(core_axis_name="c", subcore_axis_name="s"),
    out_type=jax.ShapeDtypeStruct((N_ATOMS, EMB), jnp.float32),
    scratch_types=[
        pltpu.VMEM((_SLOTS, CHUNK), jnp.int32),
        [pltpu.VMEM((CHUNK, EMB), jnp.float32)] * _DEPTH,
        pltpu.VMEM_SHARED((VOCAB, EMB), jnp.float32),
        pltpu.SemaphoreType.DMA,
        [pltpu.SemaphoreType.DMA] * _DEPTH,
        [pltpu.SemaphoreType.DMA] * _DEPTH,
    ],
)
def _emb_kernel(z_hbm, w_hbm, out_hbm, idx_v, rows, w_sh, isem, gsem, ssem):
    wid = lax.axis_index("s") * NC + lax.axis_index("c")

    @pl.when(lax.axis_index("s") == 0)
    def _():
        pltpu.sync_copy(w_hbm, w_sh)

    plsc.subcore_barrier()

    def base_of(k):
        c = jnp.minimum(wid + k * NW, _N_CHUNKS - 1)
        return pl.multiple_of(jnp.minimum(c * CHUNK, _LAST_BASE), 8)

    def when_owned(k, fn):
        if k == _SLOTS - 1:
            pl.when(wid < _FULL_W)(fn)
        else:
            fn()

    def prefetch(k):
        pltpu.async_copy(z_hbm.at[pl.ds(base_of(k), CHUNK)], idx_v.at[k],
                         isem)

    for k in range(_SLOTS):
        when_owned(k, functools.partial(prefetch, k))

    def gather(k):
        b = k % _DEPTH
        pltpu.make_async_copy(z_hbm.at[pl.ds(base_of(k), CHUNK)],
                              idx_v.at[k], isem).wait()
        for j in range(CHUNK // L):
            sl = pl.ds(j * L, L)
            idx_v[k, sl] = idx_v[k, sl] - 1
        pltpu.async_copy(w_sh.at[idx_v.at[k]], rows[b], gsem[b])

    def scatter(k):
        b = k % _DEPTH
        pltpu.make_async_copy(w_sh.at[idx_v.at[k]], rows[b],
                              gsem[b]).wait()
        pltpu.async_copy(rows[b], out_hbm.at[pl.ds(base_of(k), CHUNK)],
                         ssem[b])

    def wait_scatter(k):
        b = k % _DEPTH
        pltpu.make_async_copy(rows[b], out_hbm.at[pl.ds(base_of(k), CHUNK)],
                              ssem[b]).wait()

    for k in range(_SLOTS):
        if k >= _DEPTH:
            wait_scatter(k - _DEPTH)
        when_owned(k, functools.partial(gather, k))
        if k >= 1:
            when_owned(k - 1, functools.partial(scatter, k - 1))
    when_owned(_SLOTS - 1, functools.partial(scatter, _SLOTS - 1))
    for k in range(_SLOTS - _DEPTH, _SLOTS):
        when_owned(k, functools.partial(wait_scatter, k))


def kernel(Z, W):
    return _emb_kernel(Z, W)
